# Initial kernel scaffold; baseline (speedup 1.0000x reference)
import jax
import jax.numpy as jnp
from jax.experimental import pallas as pl


def kernel(x, edge_index, idx_subset, W1, b1, W2, b2, Wih0, Whh0, bih0, bhh0, Wih1, Whh1, bih1, bhh1, Wh, bh):
    raise NotImplementedError("write your pallas kernel here")


# speedup vs baseline: 26.3752x; 26.3752x over previous
import functools

import jax
import jax.numpy as jnp
from jax import lax
from jax.experimental import pallas as pl
from jax.experimental.pallas import tpu as pltpu
from jax.experimental.pallas import tpu_sc as plsc

F32 = jnp.float32
I32 = jnp.int32

NC, NS = 2, 16
NW = NC * NS
NPAD = 10240
ROWS_PER_TILE = NPAD // NS
TN = 1024


def _mesh():
    return plsc.VectorSubcoreMesh(core_axis_name="c", subcore_axis_name="s")


def _worker_id():
    return lax.axis_index("c") * NS + lax.axis_index("s")


def _deg_body(EB, dst_hbm, out_hbm, dstv, unit, zbuf, acc):
    c = lax.axis_index("c")
    s = lax.axis_index("s")
    w = c * NS + s
    pltpu.sync_copy(dst_hbm.at[w], dstv)

    io = lax.iota(I32, 16)
    e0 = jnp.where(io == 0, 1.0, 0.0).astype(F32)
    z16 = jnp.zeros((16,), F32)

    def fill(i, _):
        unit[i] = e0
        zbuf[i] = z16
        return 0

    lax.fori_loop(0, 128, fill, 0)

    row0 = s * ROWS_PER_TILE
    for k in range(ROWS_PER_TILE // 128):
        pltpu.sync_copy(zbuf, acc.at[pl.ds(row0 + 128 * k, 128)])
    plsc.subcore_barrier()

    def ebody(j, _):
        pltpu.sync_copy(unit, acc.at[dstv.at[j]], add=True)
        return 0

    lax.fori_loop(0, EB, ebody, 0)
    plsc.subcore_barrier()

    for k in range(ROWS_PER_TILE // 128):
        rows = pl.ds(row0 + 128 * k, 128)
        pltpu.sync_copy(acc.at[rows], zbuf)
        pltpu.sync_copy(zbuf, out_hbm.at[c, rows])


def _make_deg(EB):
    return pl.kernel(
        functools.partial(_deg_body, EB),
        out_type=jax.ShapeDtypeStruct((NC, NPAD, 16), F32),
        mesh=_mesh(),
        scratch_types=[
            pltpu.VMEM((EB, 128), I32),
            pltpu.VMEM((128, 16), F32),
            pltpu.VMEM((128, 16), F32),
            pltpu.VMEM_SHARED((NPAD, 16), F32),
        ],
    )


def _scatter_body(T, N, EB, src_hbm, dst_hbm, ysf_hbm, out_hbm,
                  srcv, dstv, bufa, acc):
    c = lax.axis_index("c")
    s = lax.axis_index("s")
    w = c * NS + s
    pltpu.sync_copy(src_hbm.at[w], srcv)
    pltpu.sync_copy(dst_hbm.at[w], dstv)

    z16 = jnp.zeros((16,), F32)
    nlane = bufa.shape[1] // 16

    def zrow(i, _):
        for k in range(nlane):
            bufa[i, pl.ds(16 * k, 16)] = z16
        return 0

    row0 = s * ROWS_PER_TILE
    for t in range(T):
        lax.fori_loop(0, 128, zrow, 0)
        for k in range(ROWS_PER_TILE // 128):
            pltpu.sync_copy(bufa, acc.at[pl.ds(row0 + 128 * k, 128)])
        plsc.subcore_barrier()

        def ebody(j, _):
            pltpu.sync_copy(ysf_hbm.at[srcv.at[j]], bufa)
            pltpu.sync_copy(bufa, acc.at[dstv.at[j]], add=True)
            return 0

        lax.fori_loop(0, EB, ebody, 0)
        plsc.subcore_barrier()

        for k in range(ROWS_PER_TILE // 128):
            rows = pl.ds(row0 + 128 * k, 128)
            pltpu.sync_copy(acc.at[rows], bufa)
            pltpu.sync_copy(bufa, out_hbm.at[c, t, rows])

        if t < T - 1:
            def ubody(j, _):
                for k in range(8):
                    sl = pl.ds(16 * k, 16)
                    srcv[j, sl] = srcv[j, sl] + N
                return 0

            lax.fori_loop(0, EB, ubody, 0)


def _make_scatter(T, N, EB, FEAT):
    return pl.kernel(
        functools.partial(_scatter_body, T, N, EB),
        out_type=jax.ShapeDtypeStruct((NC, T, NPAD, FEAT), F32),
        mesh=_mesh(),
        scratch_types=[
            pltpu.VMEM((EB, 128), I32),
            pltpu.VMEM((EB, 128), I32),
            pltpu.VMEM((128, FEAT), F32),
            pltpu.VMEM_SHARED((NPAD, FEAT), F32),
        ],
    )


def _gather_body(T, N, BW, p2_hbm, y2f_hbm, idx_hbm,
                 seq_hbm, idxv, idxs, bufy, bufa, bufb):
    c = lax.axis_index("c")
    s = lax.axis_index("s")
    w = c * NS + s
    pltpu.sync_copy(idx_hbm.at[pl.ds(w * BW, BW)], idxv)

    nchunk = BW // 16
    nlane = bufy.shape[1] // 16

    def shift(off):
        def body(k, _):
            sl = pl.ds(16 * k, 16)
            idxs[sl] = idxv[sl] + off
            return 0
        lax.fori_loop(0, nchunk, body, 0)

    for t in range(T):
        shift(jnp.int32(t * N))
        pltpu.sync_copy(y2f_hbm.at[idxs], bufy)
        shift(jnp.int32(t * NPAD))
        pltpu.sync_copy(p2_hbm.at[idxs], bufa)
        shift(jnp.int32((T + t) * NPAD))
        pltpu.sync_copy(p2_hbm.at[idxs], bufb)

        def fin(i, _):
            for k in range(nlane):
                sl = pl.ds(16 * k, 16)
                bufy[i, sl] = bufy[i, sl] + bufa[i, sl] + bufb[i, sl]
            return 0

        lax.fori_loop(0, BW, fin, 0)
        pltpu.sync_copy(bufy, seq_hbm.at[t, pl.ds(w * BW, BW)])


def _make_gather(T, N, B, FEAT):
    BW = B // NW
    return pl.kernel(
        functools.partial(_gather_body, T, N, BW),
        out_type=jax.ShapeDtypeStruct((T, B, FEAT), F32),
        mesh=_mesh(),
        scratch_types=[
            pltpu.VMEM((BW,), I32),
            pltpu.VMEM((BW,), I32),
            pltpu.VMEM((BW, FEAT), F32),
            pltpu.VMEM((BW, FEAT), F32),
            pltpu.VMEM((BW, FEAT), F32),
        ],
    )


def _dsr_body(BW, dinvrep_hbm, idx_hbm, dsr_hbm, idxv, drep):
    c = lax.axis_index("c")
    s = lax.axis_index("s")
    w = c * NS + s
    pltpu.sync_copy(idx_hbm.at[pl.ds(w * BW, BW)], idxv)
    pltpu.sync_copy(dinvrep_hbm.at[idxv], drep)
    pltpu.sync_copy(drep, dsr_hbm.at[pl.ds(w * BW, BW)])


def _make_dsr(B):
    BW = B // NW
    return pl.kernel(
        functools.partial(_dsr_body, BW),
        out_type=jax.ShapeDtypeStruct((B, 128), F32),
        mesh=_mesh(),
        scratch_types=[
            pltpu.VMEM((BW,), I32),
            pltpu.VMEM((BW, 128), F32),
        ],
    )


def _k1_body(x_ref, degp_ref, w1_ref, ys_ref, dinv_ref, dinvrep_ref):
    deg = jnp.sum(degp_ref[...], axis=(0, 2)) + 1.0
    dinv = lax.rsqrt(deg)
    y = jnp.dot(x_ref[0], w1_ref[...].T, preferred_element_type=F32)
    ys_ref[0] = y * dinv[:, None]
    dinv_ref[0] = dinv
    dinvrep_ref[...] = jnp.broadcast_to(dinv[:, None], (dinv.shape[0], 128))


def _k3_body(p_ref, ys_ref, dinv_ref, w2_ref, b1_ref, y2_ref):
    u = p_ref[0, 0] + p_ref[1, 0] + ys_ref[0]
    dv = dinv_ref[0][:, None]
    pre = u * dv + b1_ref[...]
    h = jnp.where(pre >= 0, pre, 0.01 * pre)
    y2_ref[0] = jnp.dot(h, w2_ref[...].T, preferred_element_type=F32) * dv


def _k6_body(T, BT, s_ref, dsr_ref, b2_ref, wih0_ref, whh0_ref, bs0_ref,
             wih1_ref, whh1_ref, bs1_ref, wh_ref, bh_ref, out_ref, outs_ref):
    dsr = dsr_ref[...]
    b2row = b2_ref[...]
    wih0 = wih0_ref[...]
    whh0 = whh0_ref[...]
    bs0 = bs0_ref[...]
    wih1 = wih1_ref[...]
    whh1 = whh1_ref[...]
    bs1 = bs1_ref[...]
    H = whh0.shape[1]

    def cell(xt, h, c, wih, whh, bs):
        gates = (jnp.dot(xt, wih.T, preferred_element_type=F32)
                 + jnp.dot(h, whh.T, preferred_element_type=F32) + bs)
        i = gates[:, 0:H]
        f = gates[:, H:2 * H]
        g = gates[:, 2 * H:3 * H]
        o = gates[:, 3 * H:4 * H]
        c2 = jax.nn.sigmoid(f) * c + jax.nn.sigmoid(i) * jnp.tanh(g)
        h2 = jax.nn.sigmoid(o) * jnp.tanh(c2)
        return h2, c2

    z = jnp.zeros((BT, H), F32)

    def step0(t, hc):
        h, c = hc
        xt = s_ref[pl.ds(t, 1)][0] * dsr + b2row
        h2, c2 = cell(xt, h, c, wih0, whh0, bs0)
        outs_ref[pl.ds(t, 1)] = h2[None]
        return (h2, c2)

    h0, c0 = lax.fori_loop(0, T, step0, (z, z))

    def step1(t, hc):
        h, c = hc
        xt = outs_ref[pl.ds(t, 1)][0]
        return cell(xt, h, c, wih1, whh1, bs1)

    h1, c1 = lax.fori_loop(0, T, step1, (z, z))

    io_r = lax.broadcasted_iota(jnp.int32, (BT // 2, BT), 0)
    io_c = lax.broadcasted_iota(jnp.int32, (BT // 2, BT), 1)
    pe = (io_c == 2 * io_r).astype(F32)
    po = (io_c == 2 * io_r + 1).astype(F32)

    def dei(m, a):
        return jnp.dot(m, a, preferred_element_type=F32)

    g0 = jnp.concatenate([dei(pe, h0), dei(po, h0), dei(pe, c0), dei(po, c0)],
                         axis=1)
    g1 = jnp.concatenate([dei(pe, h1), dei(po, h1), dei(pe, c1), dei(po, c1)],
                         axis=1)
    wh = wh_ref[...]
    bhs = bh_ref[0, 0]
    out_ref[0, 0] = jnp.sum(g0 * wh, axis=1) + bhs
    out_ref[0, 1] = jnp.sum(g1 * wh, axis=1) + bhs


def kernel(x, edge_index, idx_subset, W1, b1, W2, b2, Wih0, Whh0, bih0, bhh0,
           Wih1, Whh1, bih1, bhh1, Wh, bh):
    T, N, F = x.shape
    E = edge_index.shape[1]
    B = idx_subset.shape[0]
    H1 = W1.shape[0]
    LAT = W2.shape[0]
    HID = Whh0.shape[1]

    epw = -(-E // NW)
    EB = -(-epw // 128)
    EPAD = EB * 128 * NW
    padn = EPAD - E
    ar = jnp.arange(padn, dtype=I32)
    srcp = jnp.concatenate([edge_index[0], (ar * 7919) % N])
    dstp = jnp.concatenate([edge_index[1], N + (ar % (NPAD - N))])
    src_w = srcp.reshape(NW, EB, 128)
    dst_w = dstp.reshape(NW, EB, 128)

    degp = _make_deg(EB)(dst_w)

    grid1 = (T, -(-N // TN))
    ys, dinv2d, dinvrep = pl.pallas_call(
        _k1_body,
        grid=grid1,
        in_specs=[
            pl.BlockSpec((1, TN, F), lambda t, n: (t, n, 0)),
            pl.BlockSpec((NC, TN, 16), lambda t, n: (0, n, 0)),
            pl.BlockSpec((H1, F), lambda t, n: (0, 0)),
        ],
        out_specs=[
            pl.BlockSpec((1, TN, H1), lambda t, n: (t, n, 0)),
            pl.BlockSpec((1, TN), lambda t, n: (0, n)),
            pl.BlockSpec((TN, 128), lambda t, n: (n, 0)),
        ],
        out_shape=[
            jax.ShapeDtypeStruct((T, N, H1), F32),
            jax.ShapeDtypeStruct((1, N), F32),
            jax.ShapeDtypeStruct((N, 128), F32),
        ],
    )(x, degp, W1)

    scat = _make_scatter(T, N, EB, H1)
    p1 = scat(src_w, dst_w, ys.reshape(T * N, H1))

    y2s = pl.pallas_call(
        _k3_body,
        grid=grid1,
        in_specs=[
            pl.BlockSpec((NC, 1, TN, H1), lambda t, n: (0, t, n, 0)),
            pl.BlockSpec((1, TN, H1), lambda t, n: (t, n, 0)),
            pl.BlockSpec((1, TN), lambda t, n: (0, n)),
            pl.BlockSpec((LAT, H1), lambda t, n: (0, 0)),
            pl.BlockSpec((1, H1), lambda t, n: (0, 0)),
        ],
        out_specs=pl.BlockSpec((1, TN, LAT), lambda t, n: (t, n, 0)),
        out_shape=jax.ShapeDtypeStruct((T, N, LAT), F32),
    )(p1, ys, dinv2d, W2, b1.reshape(1, H1))

    p2 = scat(src_w, dst_w, y2s.reshape(T * N, LAT))

    ssum = _make_gather(T, N, B, LAT)(
        p2.reshape(NC * T * NPAD, LAT),
        y2s.reshape(T * N, LAT),
        idx_subset,
    )
    dsr = _make_dsr(B)(dinvrep, idx_subset)

    BT = 256
    nb = B // BT
    out8 = pl.pallas_call(
        functools.partial(_k6_body, T, BT),
        grid=(nb,),
        in_specs=[
            pl.BlockSpec((T, BT, LAT), lambda b: (0, b, 0)),
            pl.BlockSpec((BT, 128), lambda b: (b, 0)),
            pl.BlockSpec((1, LAT), lambda b: (0, 0)),
            pl.BlockSpec((4 * HID, LAT), lambda b: (0, 0)),
            pl.BlockSpec((4 * HID, HID), lambda b: (0, 0)),
            pl.BlockSpec((1, 4 * HID), lambda b: (0, 0)),
            pl.BlockSpec((4 * HID, HID), lambda b: (0, 0)),
            pl.BlockSpec((4 * HID, HID), lambda b: (0, 0)),
            pl.BlockSpec((1, 4 * HID), lambda b: (0, 0)),
            pl.BlockSpec((1, 4 * HID), lambda b: (0, 0)),
            pl.BlockSpec((1, 1), lambda b: (0, 0)),
        ],
        out_specs=pl.BlockSpec((1, 2, BT // 2), lambda b: (b, 0, 0)),
        out_shape=jax.ShapeDtypeStruct((nb, 2, BT // 2), F32),
        scratch_shapes=[pltpu.VMEM((T, BT, HID), F32)],
    )(
        ssum,
        dsr,
        b2.reshape(1, LAT),
        Wih0, Whh0, (bih0 + bhh0).reshape(1, 4 * HID),
        Wih1, Whh1, (bih1 + bhh1).reshape(1, 4 * HID),
        Wh, bh.reshape(1, 1),
    )
    return out8.transpose(1, 0, 2).reshape(B)


# speedup vs baseline: 34.2301x; 1.2978x over previous
import functools

import jax
import jax.numpy as jnp
from jax import lax
from jax.experimental import pallas as pl
from jax.experimental.pallas import tpu as pltpu
from jax.experimental.pallas import tpu_sc as plsc

F32 = jnp.float32
I32 = jnp.int32

NC, NS = 2, 16
NW = NC * NS
NPAD = 10240
ROWS_PER_TILE = NPAD // NS
TN = 1024


def _mesh():
    return plsc.VectorSubcoreMesh(core_axis_name="c", subcore_axis_name="s")


def _worker_id():
    return lax.axis_index("c") * NS + lax.axis_index("s")


def _deg_body(EB, dst_hbm, out_hbm, dstv, unit, zbuf, acc):
    c = lax.axis_index("c")
    s = lax.axis_index("s")
    w = c * NS + s
    pltpu.sync_copy(dst_hbm.at[w], dstv)

    io = lax.iota(I32, 16)
    e0 = jnp.where(io == 0, 1.0, 0.0).astype(F32)
    z16 = jnp.zeros((16,), F32)

    def fill(i, _):
        unit[i] = e0
        zbuf[i] = z16
        return 0

    lax.fori_loop(0, 128, fill, 0)

    row0 = s * ROWS_PER_TILE
    for k in range(ROWS_PER_TILE // 128):
        pltpu.sync_copy(zbuf, acc.at[pl.ds(row0 + 128 * k, 128)])
    plsc.subcore_barrier()

    def ebody(j, _):
        pltpu.sync_copy(unit, acc.at[dstv.at[j]], add=True)
        return 0

    lax.fori_loop(0, EB, ebody, 0)
    plsc.subcore_barrier()

    for k in range(ROWS_PER_TILE // 128):
        rows = pl.ds(row0 + 128 * k, 128)
        pltpu.sync_copy(acc.at[rows], zbuf)
        pltpu.sync_copy(zbuf, out_hbm.at[c, rows])


def _make_deg(EB):
    return pl.kernel(
        functools.partial(_deg_body, EB),
        out_type=jax.ShapeDtypeStruct((NC, NPAD, 16), F32),
        mesh=_mesh(),
        scratch_types=[
            pltpu.VMEM((EB, 128), I32),
            pltpu.VMEM((128, 16), F32),
            pltpu.VMEM((128, 16), F32),
            pltpu.VMEM_SHARED((NPAD, 16), F32),
        ],
    )


def _scatter_body(T, N, EB, src_hbm, dst_hbm, ysf_hbm, out_hbm,
                  srcv, dstrow0, dstrow1, bufa, bufg1, acc,
                  semA, semB, semD0, semD1):
    c = lax.axis_index("c")
    s = lax.axis_index("s")
    w = c * NS + s
    pltpu.sync_copy(src_hbm.at[w], srcv)

    z16 = jnp.zeros((16,), F32)
    nlane = bufa.shape[1] // 16
    EBH = EB // 2

    def zrow(i, _):
        for k in range(nlane):
            bufa[i, pl.ds(16 * k, 16)] = z16
        return 0

    row0 = s * ROWS_PER_TILE
    for t in range(T):
        lax.fori_loop(0, 128, zrow, 0)
        for k in range(ROWS_PER_TILE // 128):
            pltpu.sync_copy(bufa, acc.at[pl.ds(row0 + 128 * k, 128)])
        plsc.subcore_barrier()

        pltpu.async_copy(ysf_hbm.at[srcv.at[0]], bufa, semA)

        def ebody(g, _):
            ja = 2 * g
            jb = 2 * g + 1
            gnx = jnp.minimum(ja + 2, EB - 2)
            pltpu.async_copy(dst_hbm.at[w, ja], dstrow0, semD0)
            pltpu.async_copy(dst_hbm.at[w, jb], dstrow1, semD1)
            pltpu.make_async_copy(ysf_hbm.at[srcv.at[ja]], bufa, semA).wait()
            pltpu.async_copy(ysf_hbm.at[srcv.at[jb]], bufg1, semB)
            pltpu.make_async_copy(dst_hbm.at[w, ja], dstrow0, semD0).wait()
            pltpu.sync_copy(bufa, acc.at[dstrow0], add=True)
            pltpu.make_async_copy(ysf_hbm.at[srcv.at[jb]], bufg1, semB).wait()
            pltpu.async_copy(ysf_hbm.at[srcv.at[gnx]], bufa, semA)
            pltpu.make_async_copy(dst_hbm.at[w, jb], dstrow1, semD1).wait()
            pltpu.sync_copy(bufg1, acc.at[dstrow1], add=True)
            return 0

        lax.fori_loop(0, EBH, ebody, 0)
        pltpu.make_async_copy(ysf_hbm.at[srcv.at[0]], bufa, semA).wait()
        plsc.subcore_barrier()

        for k in range(ROWS_PER_TILE // 128):
            rows = pl.ds(row0 + 128 * k, 128)
            pltpu.sync_copy(acc.at[rows], bufa)
            pltpu.sync_copy(bufa, out_hbm.at[c, t, rows])

        if t < T - 1:
            def ubody(j, _):
                for k in range(8):
                    sl = pl.ds(16 * k, 16)
                    srcv[j, sl] = srcv[j, sl] + N
                return 0

            lax.fori_loop(0, EB, ubody, 0)


def _make_scatter(T, N, EB, FEAT):
    return pl.kernel(
        functools.partial(_scatter_body, T, N, EB),
        out_type=jax.ShapeDtypeStruct((NC, T, NPAD, FEAT), F32),
        mesh=_mesh(),
        scratch_types=[
            pltpu.VMEM((EB, 128), I32),
            pltpu.VMEM((128,), I32),
            pltpu.VMEM((128,), I32),
            pltpu.VMEM((128, FEAT), F32),
            pltpu.VMEM((128, FEAT), F32),
            pltpu.VMEM_SHARED((NPAD, FEAT), F32),
            pltpu.SemaphoreType.DMA,
            pltpu.SemaphoreType.DMA,
            pltpu.SemaphoreType.DMA,
            pltpu.SemaphoreType.DMA,
        ],
    )


def _gather_body(T, N, BW, p2_hbm, y2f_hbm, idx_hbm,
                 seq_hbm, idxv, idxs, bufy, bufa, bufb):
    c = lax.axis_index("c")
    s = lax.axis_index("s")
    w = c * NS + s
    pltpu.sync_copy(idx_hbm.at[pl.ds(w * BW, BW)], idxv)

    nchunk = BW // 16
    nlane = bufy.shape[1] // 16

    def shift(off):
        def body(k, _):
            sl = pl.ds(16 * k, 16)
            idxs[sl] = idxv[sl] + off
            return 0
        lax.fori_loop(0, nchunk, body, 0)

    for t in range(T):
        shift(jnp.int32(t * N))
        pltpu.sync_copy(y2f_hbm.at[idxs], bufy)
        shift(jnp.int32(t * NPAD))
        pltpu.sync_copy(p2_hbm.at[idxs], bufa)
        shift(jnp.int32((T + t) * NPAD))
        pltpu.sync_copy(p2_hbm.at[idxs], bufb)

        def fin(i, _):
            for k in range(nlane):
                sl = pl.ds(16 * k, 16)
                bufy[i, sl] = bufy[i, sl] + bufa[i, sl] + bufb[i, sl]
            return 0

        lax.fori_loop(0, BW, fin, 0)
        pltpu.sync_copy(bufy, seq_hbm.at[t, pl.ds(w * BW, BW)])


def _make_gather(T, N, B, FEAT):
    BW = B // NW
    return pl.kernel(
        functools.partial(_gather_body, T, N, BW),
        out_type=jax.ShapeDtypeStruct((T, B, FEAT), F32),
        mesh=_mesh(),
        scratch_types=[
            pltpu.VMEM((BW,), I32),
            pltpu.VMEM((BW,), I32),
            pltpu.VMEM((BW, FEAT), F32),
            pltpu.VMEM((BW, FEAT), F32),
            pltpu.VMEM((BW, FEAT), F32),
        ],
    )


def _dsr_body(BW, dinvrep_hbm, idx_hbm, dsr_hbm, idxv, drep):
    c = lax.axis_index("c")
    s = lax.axis_index("s")
    w = c * NS + s
    pltpu.sync_copy(idx_hbm.at[pl.ds(w * BW, BW)], idxv)
    pltpu.sync_copy(dinvrep_hbm.at[idxv], drep)
    pltpu.sync_copy(drep, dsr_hbm.at[pl.ds(w * BW, BW)])


def _make_dsr(B):
    BW = B // NW
    return pl.kernel(
        functools.partial(_dsr_body, BW),
        out_type=jax.ShapeDtypeStruct((B, 128), F32),
        mesh=_mesh(),
        scratch_types=[
            pltpu.VMEM((BW,), I32),
            pltpu.VMEM((BW, 128), F32),
        ],
    )


def _k1_body(x_ref, degp_ref, w1_ref, ys_ref, dinv_ref, dinvrep_ref):
    deg = jnp.sum(degp_ref[...], axis=(0, 2)) + 1.0
    dinv = lax.rsqrt(deg)
    y = jnp.dot(x_ref[0], w1_ref[...].T, preferred_element_type=F32)
    ys_ref[0] = y * dinv[:, None]
    dinv_ref[0] = dinv
    dinvrep_ref[...] = jnp.broadcast_to(dinv[:, None], (dinv.shape[0], 128))


def _k3_body(p_ref, ys_ref, dinv_ref, w2_ref, b1_ref, y2_ref):
    u = p_ref[0, 0] + p_ref[1, 0] + ys_ref[0]
    dv = dinv_ref[0][:, None]
    pre = u * dv + b1_ref[...]
    h = jnp.where(pre >= 0, pre, 0.01 * pre)
    y2_ref[0] = jnp.dot(h, w2_ref[...].T, preferred_element_type=F32) * dv


def _k6_body(T, BT, s_ref, dsr_ref, b2_ref, wih0_ref, whh0_ref, bs0_ref,
             wih1_ref, whh1_ref, bs1_ref, wh_ref, bh_ref, out_ref, outs_ref):
    dsr = dsr_ref[...]
    b2row = b2_ref[...]
    wih0 = wih0_ref[...]
    whh0 = whh0_ref[...]
    bs0 = bs0_ref[...]
    wih1 = wih1_ref[...]
    whh1 = whh1_ref[...]
    bs1 = bs1_ref[...]
    H = whh0.shape[1]

    def cell(xt, h, c, wih, whh, bs):
        gates = (jnp.dot(xt, wih.T, preferred_element_type=F32)
                 + jnp.dot(h, whh.T, preferred_element_type=F32) + bs)
        i = gates[:, 0:H]
        f = gates[:, H:2 * H]
        g = gates[:, 2 * H:3 * H]
        o = gates[:, 3 * H:4 * H]
        c2 = jax.nn.sigmoid(f) * c + jax.nn.sigmoid(i) * jnp.tanh(g)
        h2 = jax.nn.sigmoid(o) * jnp.tanh(c2)
        return h2, c2

    z = jnp.zeros((BT, H), F32)

    def step0(t, hc):
        h, c = hc
        xt = s_ref[pl.ds(t, 1)][0] * dsr + b2row
        h2, c2 = cell(xt, h, c, wih0, whh0, bs0)
        outs_ref[pl.ds(t, 1)] = h2[None]
        return (h2, c2)

    h0, c0 = lax.fori_loop(0, T, step0, (z, z))

    def step1(t, hc):
        h, c = hc
        xt = outs_ref[pl.ds(t, 1)][0]
        return cell(xt, h, c, wih1, whh1, bs1)

    h1, c1 = lax.fori_loop(0, T, step1, (z, z))

    io_r = lax.broadcasted_iota(jnp.int32, (BT // 2, BT), 0)
    io_c = lax.broadcasted_iota(jnp.int32, (BT // 2, BT), 1)
    pe = (io_c == 2 * io_r).astype(F32)
    po = (io_c == 2 * io_r + 1).astype(F32)

    def dei(m, a):
        return jnp.dot(m, a, preferred_element_type=F32)

    g0 = jnp.concatenate([dei(pe, h0), dei(po, h0), dei(pe, c0), dei(po, c0)],
                         axis=1)
    g1 = jnp.concatenate([dei(pe, h1), dei(po, h1), dei(pe, c1), dei(po, c1)],
                         axis=1)
    wh = wh_ref[...]
    bhs = bh_ref[0, 0]
    out_ref[0, 0] = jnp.sum(g0 * wh, axis=1) + bhs
    out_ref[0, 1] = jnp.sum(g1 * wh, axis=1) + bhs


def kernel(x, edge_index, idx_subset, W1, b1, W2, b2, Wih0, Whh0, bih0, bhh0,
           Wih1, Whh1, bih1, bhh1, Wh, bh):
    T, N, F = x.shape
    E = edge_index.shape[1]
    B = idx_subset.shape[0]
    H1 = W1.shape[0]
    LAT = W2.shape[0]
    HID = Whh0.shape[1]

    epw = -(-E // NW)
    EB = -(-epw // 128)
    EB = EB + (EB & 1)
    EPAD = EB * 128 * NW
    padn = EPAD - E
    ar = jnp.arange(padn, dtype=I32)
    srcp = jnp.concatenate([edge_index[0], (ar * 7919) % N])
    dstp = jnp.concatenate([edge_index[1], N + (ar % (NPAD - N))])
    src_w = srcp.reshape(NW, EB, 128)
    dst_w = dstp.reshape(NW, EB, 128)

    degp = _make_deg(EB)(dst_w)

    grid1 = (T, -(-N // TN))
    ys, dinv2d, dinvrep = pl.pallas_call(
        _k1_body,
        grid=grid1,
        in_specs=[
            pl.BlockSpec((1, TN, F), lambda t, n: (t, n, 0)),
            pl.BlockSpec((NC, TN, 16), lambda t, n: (0, n, 0)),
            pl.BlockSpec((H1, F), lambda t, n: (0, 0)),
        ],
        out_specs=[
            pl.BlockSpec((1, TN, H1), lambda t, n: (t, n, 0)),
            pl.BlockSpec((1, TN), lambda t, n: (0, n)),
            pl.BlockSpec((TN, 128), lambda t, n: (n, 0)),
        ],
        out_shape=[
            jax.ShapeDtypeStruct((T, N, H1), F32),
            jax.ShapeDtypeStruct((1, N), F32),
            jax.ShapeDtypeStruct((N, 128), F32),
        ],
    )(x, degp, W1)

    scat = _make_scatter(T, N, EB, H1)
    p1 = scat(src_w, dst_w, ys.reshape(T * N, H1))

    y2s = pl.pallas_call(
        _k3_body,
        grid=grid1,
        in_specs=[
            pl.BlockSpec((NC, 1, TN, H1), lambda t, n: (0, t, n, 0)),
            pl.BlockSpec((1, TN, H1), lambda t, n: (t, n, 0)),
            pl.BlockSpec((1, TN), lambda t, n: (0, n)),
            pl.BlockSpec((LAT, H1), lambda t, n: (0, 0)),
            pl.BlockSpec((1, H1), lambda t, n: (0, 0)),
        ],
        out_specs=pl.BlockSpec((1, TN, LAT), lambda t, n: (t, n, 0)),
        out_shape=jax.ShapeDtypeStruct((T, N, LAT), F32),
    )(p1, ys, dinv2d, W2, b1.reshape(1, H1))

    p2 = scat(src_w, dst_w, y2s.reshape(T * N, LAT))

    ssum = _make_gather(T, N, B, LAT)(
        p2.reshape(NC * T * NPAD, LAT),
        y2s.reshape(T * N, LAT),
        idx_subset,
    )
    dsr = _make_dsr(B)(dinvrep, idx_subset)

    BT = 256
    nb = B // BT
    out8 = pl.pallas_call(
        functools.partial(_k6_body, T, BT),
        grid=(nb,),
        in_specs=[
            pl.BlockSpec((T, BT, LAT), lambda b: (0, b, 0)),
            pl.BlockSpec((BT, 128), lambda b: (b, 0)),
            pl.BlockSpec((1, LAT), lambda b: (0, 0)),
            pl.BlockSpec((4 * HID, LAT), lambda b: (0, 0)),
            pl.BlockSpec((4 * HID, HID), lambda b: (0, 0)),
            pl.BlockSpec((1, 4 * HID), lambda b: (0, 0)),
            pl.BlockSpec((4 * HID, HID), lambda b: (0, 0)),
            pl.BlockSpec((4 * HID, HID), lambda b: (0, 0)),
            pl.BlockSpec((1, 4 * HID), lambda b: (0, 0)),
            pl.BlockSpec((1, 4 * HID), lambda b: (0, 0)),
            pl.BlockSpec((1, 1), lambda b: (0, 0)),
        ],
        out_specs=pl.BlockSpec((1, 2, BT // 2), lambda b: (b, 0, 0)),
        out_shape=jax.ShapeDtypeStruct((nb, 2, BT // 2), F32),
        scratch_shapes=[pltpu.VMEM((T, BT, HID), F32)],
    )(
        ssum,
        dsr,
        b2.reshape(1, LAT),
        Wih0, Whh0, (bih0 + bhh0).reshape(1, 4 * HID),
        Wih1, Whh1, (bih1 + bhh1).reshape(1, 4 * HID),
        Wh, bh.reshape(1, 1),
    )
    return out8.transpose(1, 0, 2).reshape(B)


# speedup vs baseline: 34.4360x; 1.0060x over previous
import functools

import jax
import jax.numpy as jnp
from jax import lax
from jax.experimental import pallas as pl
from jax.experimental.pallas import tpu as pltpu
from jax.experimental.pallas import tpu_sc as plsc

F32 = jnp.float32
I32 = jnp.int32

NC, NS = 2, 16
NW = NC * NS
NPAD = 10240
ROWS_PER_TILE = NPAD // NS
TN = 1024


def _mesh():
    return plsc.VectorSubcoreMesh(core_axis_name="c", subcore_axis_name="s")


def _worker_id():
    return lax.axis_index("c") * NS + lax.axis_index("s")


def _deg_body(EB, dst_hbm, out_hbm, dstv, unit, zbuf, acc):
    c = lax.axis_index("c")
    s = lax.axis_index("s")
    w = c * NS + s
    pltpu.sync_copy(dst_hbm.at[w], dstv)

    io = lax.iota(I32, 16)
    e0 = jnp.where(io == 0, 1.0, 0.0).astype(F32)
    z16 = jnp.zeros((16,), F32)

    def fill(i, _):
        unit[i] = e0
        zbuf[i] = z16
        return 0

    lax.fori_loop(0, 128, fill, 0)

    row0 = s * ROWS_PER_TILE
    for k in range(ROWS_PER_TILE // 128):
        pltpu.sync_copy(zbuf, acc.at[pl.ds(row0 + 128 * k, 128)])
    plsc.subcore_barrier()

    def ebody(j, _):
        pltpu.sync_copy(unit, acc.at[dstv.at[j]], add=True)
        return 0

    lax.fori_loop(0, EB, ebody, 0)
    plsc.subcore_barrier()

    for k in range(ROWS_PER_TILE // 128):
        rows = pl.ds(row0 + 128 * k, 128)
        pltpu.sync_copy(acc.at[rows], zbuf)
        pltpu.sync_copy(zbuf, out_hbm.at[c, rows])


def _make_deg(EB):
    return pl.kernel(
        functools.partial(_deg_body, EB),
        out_type=jax.ShapeDtypeStruct((NC, NPAD, 16), F32),
        mesh=_mesh(),
        scratch_types=[
            pltpu.VMEM((EB, 128), I32),
            pltpu.VMEM((128, 16), F32),
            pltpu.VMEM((128, 16), F32),
            pltpu.VMEM_SHARED((NPAD, 16), F32),
        ],
    )


def _scatter_body(T, N, EB, src_hbm, dst_hbm, ysf_hbm, out_hbm,
                  srcv, dstrow0, dstrow1, bufa, bufg1, acc,
                  semA, semB, semD0, semD1):
    c = lax.axis_index("c")
    s = lax.axis_index("s")
    w = c * NS + s
    pltpu.sync_copy(src_hbm.at[w], srcv)

    z16 = jnp.zeros((16,), F32)
    nlane = bufa.shape[1] // 16
    EBH = EB // 2

    def zrow(i, _):
        for k in range(nlane):
            bufa[i, pl.ds(16 * k, 16)] = z16
        return 0

    row0 = s * ROWS_PER_TILE
    for t in range(T):
        lax.fori_loop(0, 128, zrow, 0)
        for k in range(ROWS_PER_TILE // 128):
            pltpu.sync_copy(bufa, acc.at[pl.ds(row0 + 128 * k, 128)])
        plsc.subcore_barrier()

        pltpu.async_copy(ysf_hbm.at[srcv.at[0]], bufa, semA)

        def ebody(g, _):
            ja = 2 * g
            jb = 2 * g + 1
            gnx = jnp.minimum(ja + 2, EB - 2)
            pltpu.async_copy(dst_hbm.at[w, ja], dstrow0, semD0)
            pltpu.async_copy(dst_hbm.at[w, jb], dstrow1, semD1)
            pltpu.make_async_copy(ysf_hbm.at[srcv.at[ja]], bufa, semA).wait()
            pltpu.async_copy(ysf_hbm.at[srcv.at[jb]], bufg1, semB)
            pltpu.make_async_copy(dst_hbm.at[w, ja], dstrow0, semD0).wait()
            pltpu.sync_copy(bufa, acc.at[dstrow0], add=True)
            pltpu.make_async_copy(ysf_hbm.at[srcv.at[jb]], bufg1, semB).wait()
            pltpu.async_copy(ysf_hbm.at[srcv.at[gnx]], bufa, semA)
            pltpu.make_async_copy(dst_hbm.at[w, jb], dstrow1, semD1).wait()
            pltpu.sync_copy(bufg1, acc.at[dstrow1], add=True)
            return 0

        lax.fori_loop(0, EBH, ebody, 0)
        pltpu.make_async_copy(ysf_hbm.at[srcv.at[0]], bufa, semA).wait()
        plsc.subcore_barrier()

        for k in range(ROWS_PER_TILE // 128):
            rows = pl.ds(row0 + 128 * k, 128)
            pltpu.sync_copy(acc.at[rows], out_hbm.at[c, t, rows])

        if t < T - 1:
            def ubody(j, _):
                for k in range(8):
                    sl = pl.ds(16 * k, 16)
                    srcv[j, sl] = srcv[j, sl] + N
                return 0

            lax.fori_loop(0, EB, ubody, 0)


def _make_scatter(T, N, EB, FEAT):
    return pl.kernel(
        functools.partial(_scatter_body, T, N, EB),
        out_type=jax.ShapeDtypeStruct((NC, T, NPAD, FEAT), F32),
        mesh=_mesh(),
        scratch_types=[
            pltpu.VMEM((EB, 128), I32),
            pltpu.VMEM((128,), I32),
            pltpu.VMEM((128,), I32),
            pltpu.VMEM((128, FEAT), F32),
            pltpu.VMEM((128, FEAT), F32),
            pltpu.VMEM_SHARED((NPAD, FEAT), F32),
            pltpu.SemaphoreType.DMA,
            pltpu.SemaphoreType.DMA,
            pltpu.SemaphoreType.DMA,
            pltpu.SemaphoreType.DMA,
        ],
    )


def _gather_body(T, N, BW, p2_hbm, y2f_hbm, idx_hbm,
                 seq_hbm, idxv, idxs, bufy, bufa, bufb):
    c = lax.axis_index("c")
    s = lax.axis_index("s")
    w = c * NS + s
    pltpu.sync_copy(idx_hbm.at[pl.ds(w * BW, BW)], idxv)

    nchunk = BW // 16
    nlane = bufy.shape[1] // 16

    def shift(off):
        def body(k, _):
            sl = pl.ds(16 * k, 16)
            idxs[sl] = idxv[sl] + off
            return 0
        lax.fori_loop(0, nchunk, body, 0)

    for t in range(T):
        shift(jnp.int32(t * N))
        pltpu.sync_copy(y2f_hbm.at[idxs], bufy)
        shift(jnp.int32(t * NPAD))
        pltpu.sync_copy(p2_hbm.at[idxs], bufa)
        shift(jnp.int32((T + t) * NPAD))
        pltpu.sync_copy(p2_hbm.at[idxs], bufb)

        def fin(i, _):
            for k in range(nlane):
                sl = pl.ds(16 * k, 16)
                bufy[i, sl] = bufy[i, sl] + bufa[i, sl] + bufb[i, sl]
            return 0

        lax.fori_loop(0, BW, fin, 0)
        pltpu.sync_copy(bufy, seq_hbm.at[t, pl.ds(w * BW, BW)])


def _make_gather(T, N, B, FEAT):
    BW = B // NW
    return pl.kernel(
        functools.partial(_gather_body, T, N, BW),
        out_type=jax.ShapeDtypeStruct((T, B, FEAT), F32),
        mesh=_mesh(),
        scratch_types=[
            pltpu.VMEM((BW,), I32),
            pltpu.VMEM((BW,), I32),
            pltpu.VMEM((BW, FEAT), F32),
            pltpu.VMEM((BW, FEAT), F32),
            pltpu.VMEM((BW, FEAT), F32),
        ],
    )


def _dsr_body(BW, dinvrep_hbm, idx_hbm, dsr_hbm, idxv, drep):
    c = lax.axis_index("c")
    s = lax.axis_index("s")
    w = c * NS + s
    pltpu.sync_copy(idx_hbm.at[pl.ds(w * BW, BW)], idxv)
    pltpu.sync_copy(dinvrep_hbm.at[idxv], drep)
    pltpu.sync_copy(drep, dsr_hbm.at[pl.ds(w * BW, BW)])


def _make_dsr(B):
    BW = B // NW
    return pl.kernel(
        functools.partial(_dsr_body, BW),
        out_type=jax.ShapeDtypeStruct((B, 128), F32),
        mesh=_mesh(),
        scratch_types=[
            pltpu.VMEM((BW,), I32),
            pltpu.VMEM((BW, 128), F32),
        ],
    )


def _k1_body(x_ref, degp_ref, w1_ref, ys_ref, dinv_ref, dinvrep_ref):
    deg = jnp.sum(degp_ref[...], axis=(0, 2)) + 1.0
    dinv = lax.rsqrt(deg)
    y = jnp.dot(x_ref[0], w1_ref[...].T, preferred_element_type=F32)
    ys_ref[0] = y * dinv[:, None]
    dinv_ref[0] = dinv
    dinvrep_ref[...] = jnp.broadcast_to(dinv[:, None], (dinv.shape[0], 128))


def _k3_body(p_ref, ys_ref, dinv_ref, w2_ref, b1_ref, y2_ref):
    u = p_ref[0, 0] + p_ref[1, 0] + ys_ref[0]
    dv = dinv_ref[0][:, None]
    pre = u * dv + b1_ref[...]
    h = jnp.where(pre >= 0, pre, 0.01 * pre)
    y2_ref[0] = jnp.dot(h, w2_ref[...].T, preferred_element_type=F32) * dv


def _k6_body(T, BT, s_ref, dsr_ref, b2_ref, wih0_ref, whh0_ref, bs0_ref,
             wih1_ref, whh1_ref, bs1_ref, wh_ref, bh_ref, out_ref, outs_ref):
    dsr = dsr_ref[...]
    b2row = b2_ref[...]
    wih0 = wih0_ref[...]
    whh0 = whh0_ref[...]
    bs0 = bs0_ref[...]
    wih1 = wih1_ref[...]
    whh1 = whh1_ref[...]
    bs1 = bs1_ref[...]
    H = whh0.shape[1]

    def cell(xt, h, c, wih, whh, bs):
        gates = (jnp.dot(xt, wih.T, preferred_element_type=F32)
                 + jnp.dot(h, whh.T, preferred_element_type=F32) + bs)
        i = gates[:, 0:H]
        f = gates[:, H:2 * H]
        g = gates[:, 2 * H:3 * H]
        o = gates[:, 3 * H:4 * H]
        c2 = jax.nn.sigmoid(f) * c + jax.nn.sigmoid(i) * jnp.tanh(g)
        h2 = jax.nn.sigmoid(o) * jnp.tanh(c2)
        return h2, c2

    z = jnp.zeros((BT, H), F32)

    def step0(t, hc):
        h, c = hc
        xt = s_ref[pl.ds(t, 1)][0] * dsr + b2row
        h2, c2 = cell(xt, h, c, wih0, whh0, bs0)
        outs_ref[pl.ds(t, 1)] = h2[None]
        return (h2, c2)

    h0, c0 = lax.fori_loop(0, T, step0, (z, z))

    def step1(t, hc):
        h, c = hc
        xt = outs_ref[pl.ds(t, 1)][0]
        return cell(xt, h, c, wih1, whh1, bs1)

    h1, c1 = lax.fori_loop(0, T, step1, (z, z))

    io_r = lax.broadcasted_iota(jnp.int32, (BT // 2, BT), 0)
    io_c = lax.broadcasted_iota(jnp.int32, (BT // 2, BT), 1)
    pe = (io_c == 2 * io_r).astype(F32)
    po = (io_c == 2 * io_r + 1).astype(F32)

    def dei(m, a):
        return jnp.dot(m, a, preferred_element_type=F32)

    g0 = jnp.concatenate([dei(pe, h0), dei(po, h0), dei(pe, c0), dei(po, c0)],
                         axis=1)
    g1 = jnp.concatenate([dei(pe, h1), dei(po, h1), dei(pe, c1), dei(po, c1)],
                         axis=1)
    wh = wh_ref[...]
    bhs = bh_ref[0, 0]
    out_ref[0, 0] = jnp.sum(g0 * wh, axis=1) + bhs
    out_ref[0, 1] = jnp.sum(g1 * wh, axis=1) + bhs


def kernel(x, edge_index, idx_subset, W1, b1, W2, b2, Wih0, Whh0, bih0, bhh0,
           Wih1, Whh1, bih1, bhh1, Wh, bh):
    T, N, F = x.shape
    E = edge_index.shape[1]
    B = idx_subset.shape[0]
    H1 = W1.shape[0]
    LAT = W2.shape[0]
    HID = Whh0.shape[1]

    epw = -(-E // NW)
    EB = -(-epw // 128)
    EB = EB + (EB & 1)
    EPAD = EB * 128 * NW
    padn = EPAD - E
    ar = jnp.arange(padn, dtype=I32)
    srcp = jnp.concatenate([edge_index[0], (ar * 7919) % N])
    dstp = jnp.concatenate([edge_index[1], N + (ar % (NPAD - N))])
    src_w = srcp.reshape(NW, EB, 128)
    dst_w = dstp.reshape(NW, EB, 128)

    degp = _make_deg(EB)(dst_w)

    grid1 = (T, -(-N // TN))
    ys, dinv2d, dinvrep = pl.pallas_call(
        _k1_body,
        grid=grid1,
        in_specs=[
            pl.BlockSpec((1, TN, F), lambda t, n: (t, n, 0)),
            pl.BlockSpec((NC, TN, 16), lambda t, n: (0, n, 0)),
            pl.BlockSpec((H1, F), lambda t, n: (0, 0)),
        ],
        out_specs=[
            pl.BlockSpec((1, TN, H1), lambda t, n: (t, n, 0)),
            pl.BlockSpec((1, TN), lambda t, n: (0, n)),
            pl.BlockSpec((TN, 128), lambda t, n: (n, 0)),
        ],
        out_shape=[
            jax.ShapeDtypeStruct((T, N, H1), F32),
            jax.ShapeDtypeStruct((1, N), F32),
            jax.ShapeDtypeStruct((N, 128), F32),
        ],
    )(x, degp, W1)

    scat = _make_scatter(T, N, EB, H1)
    p1 = scat(src_w, dst_w, ys.reshape(T * N, H1))

    y2s = pl.pallas_call(
        _k3_body,
        grid=grid1,
        in_specs=[
            pl.BlockSpec((NC, 1, TN, H1), lambda t, n: (0, t, n, 0)),
            pl.BlockSpec((1, TN, H1), lambda t, n: (t, n, 0)),
            pl.BlockSpec((1, TN), lambda t, n: (0, n)),
            pl.BlockSpec((LAT, H1), lambda t, n: (0, 0)),
            pl.BlockSpec((1, H1), lambda t, n: (0, 0)),
        ],
        out_specs=pl.BlockSpec((1, TN, LAT), lambda t, n: (t, n, 0)),
        out_shape=jax.ShapeDtypeStruct((T, N, LAT), F32),
    )(p1, ys, dinv2d, W2, b1.reshape(1, H1))

    p2 = scat(src_w, dst_w, y2s.reshape(T * N, LAT))

    ssum = _make_gather(T, N, B, LAT)(
        p2.reshape(NC * T * NPAD, LAT),
        y2s.reshape(T * N, LAT),
        idx_subset,
    )
    dsr = _make_dsr(B)(dinvrep, idx_subset)

    BT = 256
    nb = B // BT
    out8 = pl.pallas_call(
        functools.partial(_k6_body, T, BT),
        grid=(nb,),
        in_specs=[
            pl.BlockSpec((T, BT, LAT), lambda b: (0, b, 0)),
            pl.BlockSpec((BT, 128), lambda b: (b, 0)),
            pl.BlockSpec((1, LAT), lambda b: (0, 0)),
            pl.BlockSpec((4 * HID, LAT), lambda b: (0, 0)),
            pl.BlockSpec((4 * HID, HID), lambda b: (0, 0)),
            pl.BlockSpec((1, 4 * HID), lambda b: (0, 0)),
            pl.BlockSpec((4 * HID, HID), lambda b: (0, 0)),
            pl.BlockSpec((4 * HID, HID), lambda b: (0, 0)),
            pl.BlockSpec((1, 4 * HID), lambda b: (0, 0)),
            pl.BlockSpec((1, 4 * HID), lambda b: (0, 0)),
            pl.BlockSpec((1, 1), lambda b: (0, 0)),
        ],
        out_specs=pl.BlockSpec((1, 2, BT // 2), lambda b: (b, 0, 0)),
        out_shape=jax.ShapeDtypeStruct((nb, 2, BT // 2), F32),
        scratch_shapes=[pltpu.VMEM((T, BT, HID), F32)],
    )(
        ssum,
        dsr,
        b2.reshape(1, LAT),
        Wih0, Whh0, (bih0 + bhh0).reshape(1, 4 * HID),
        Wih1, Whh1, (bih1 + bhh1).reshape(1, 4 * HID),
        Wh, bh.reshape(1, 1),
    )
    return out8.transpose(1, 0, 2).reshape(B)
